# baseline (speedup 1.0000x reference)
import functools

import jax
import jax.numpy as jnp
from jax.experimental import pallas as pl
from jax.experimental.pallas import tpu as pltpu
from jax.experimental.pallas import tpu_sc as plsc

N, E, G = 4096, 8192, 128
EBLK = 256
NBLK = 256

_NC, _NS = 2, 16
_NW = _NC * _NS
_W = E // _NW
_ROWS = N // _NS

_HI = jax.lax.Precision.HIGHEST


def _silu(v):
    return v * jax.nn.sigmoid(v)


def _mm(a, b, precision=_HI):
    return jnp.dot(a, b, preferred_element_type=jnp.float32,
                   precision=precision)


def _sc_mesh():
    return plsc.VectorSubcoreMesh(core_axis_name="c", subcore_axis_name="s")


def _sc_gather_body(x_hbm, idx_hbm, o_hbm, ibuf, vbuf, *, w):
    core = jax.lax.axis_index("c")
    sid = jax.lax.axis_index("s")
    wid = core * _NS + sid
    pltpu.sync_copy(idx_hbm.at[wid], ibuf)
    pltpu.sync_copy(x_hbm.at[ibuf.at[0]], vbuf)
    pltpu.sync_copy(vbuf, o_hbm.at[pl.ds(wid * w, w)])


def _gather(x_full, idx3):
    w = idx3.shape[2]
    k = pl.kernel(
        functools.partial(_sc_gather_body, w=w),
        out_type=jax.ShapeDtypeStruct((_NW * w, 128), jnp.float32),
        mesh=_sc_mesh(),
        scratch_types=[pltpu.VMEM((1, w), jnp.int32),
                       pltpu.VMEM((w, 128), jnp.float32)],
    )
    return k(x_full, idx3)


def _sc_scatter_body(msg_hbm, idx_hbm, init_hbm, o_hbm, acc, ibuf, vbuf, *, w):
    core = jax.lax.axis_index("c")
    sid = jax.lax.axis_index("s")

    @pl.when(sid == 0)
    def _():
        pltpu.sync_copy(init_hbm.at[core], acc)

    plsc.subcore_barrier()
    wid = core * _NS + sid
    pltpu.sync_copy(idx_hbm.at[wid], ibuf)
    pltpu.sync_copy(msg_hbm.at[pl.ds(wid * w, w)], vbuf)
    pltpu.sync_copy(vbuf, acc.at[ibuf.at[0]], add=True)
    plsc.subcore_barrier()
    pltpu.sync_copy(acc.at[pl.ds(sid * _ROWS, _ROWS)],
                    o_hbm.at[core].at[pl.ds(sid * _ROWS, _ROWS)])


def _scatter_add(msg, idx3, init):
    w = idx3.shape[2]
    k = pl.kernel(
        functools.partial(_sc_scatter_body, w=w),
        out_type=jax.ShapeDtypeStruct((_NC, N, 128), jnp.float32),
        mesh=_sc_mesh(),
        scratch_types=[pltpu.VMEM_SHARED((N, 128), jnp.float32),
                       pltpu.VMEM((1, w), jnp.int32),
                       pltpu.VMEM((w, 128), jnp.float32)],
    )
    return k(msg, idx3, init)


def _cnt_kernel(dst_ref, out_ref):
    nb = pl.program_id(0)
    row = jax.lax.broadcasted_iota(jnp.int32, (NBLK, E), 0) + nb * NBLK
    eq = (row == dst_ref[0, :][None, :]).astype(jnp.float32)
    out_ref[...] = jnp.sum(eq, axis=1, keepdims=True)


def _conv_kernel(xs_ref, ea_ref, w1_ref, b1_ref, wf_ref, msg_ref, *, nin):
    h = _silu(_mm(ea_ref[...], w1_ref[...]) + b1_ref[0, :][None, :])
    parr = _mm(h, wf_ref[...], jax.lax.Precision.DEFAULT)
    xs = xs_ref[...]
    msg = xs[:, 0:1] * parr[:, 0:128]
    for i in range(1, nin):
        msg += xs[:, i:i + 1] * parr[:, i * 128:(i + 1) * 128]
    msg_ref[...] = msg


def _conv_call(xs, edge_attr, w1, b1, wf, nin):
    n_eb = xs.shape[0] // EBLK
    const = lambda shape: pl.BlockSpec(shape, lambda eb: (0,) * len(shape))
    kfn = functools.partial(_conv_kernel, nin=nin)
    return pl.pallas_call(
        kfn,
        grid=(n_eb,),
        in_specs=[
            pl.BlockSpec((EBLK, 128), lambda eb: (eb, 0)),
            pl.BlockSpec((EBLK, 3), lambda eb: (eb, 0)),
            const(w1.shape), const(b1.shape), const(wf.shape),
        ],
        out_specs=pl.BlockSpec((EBLK, 128), lambda eb: (eb, 0)),
        out_shape=jax.ShapeDtypeStruct((xs.shape[0], 128), jnp.float32),
        compiler_params=pltpu.CompilerParams(
            dimension_semantics=("arbitrary",)),
    )(xs, edge_attr, w1, b1, wf)


def _epi_kernel(p_ref, cnt_ref, x_ref, root_ref, bias_ref, out_ref):
    agg = (p_ref[0] + p_ref[1]) / cnt_ref[...]
    out_ref[...] = _silu(
        agg + _mm(x_ref[...], root_ref[...]) + bias_ref[0, :][None, :])


def _epilogue(partials, cnt, x_full, root, bias):
    return pl.pallas_call(
        _epi_kernel,
        in_specs=[
            pl.BlockSpec((_NC, N, 128), lambda: (0, 0, 0)),
            pl.BlockSpec((N, 1), lambda: (0, 0)),
            pl.BlockSpec(x_full.shape, lambda: (0, 0)),
            pl.BlockSpec((128, 128), lambda: (0, 0)),
            pl.BlockSpec((1, 128), lambda: (0, 0)),
        ],
        out_specs=pl.BlockSpec((N, 128), lambda: (0, 0)),
        out_shape=jax.ShapeDtypeStruct((N, 128), jnp.float32),
    )(partials, cnt, x_full, root, bias)


def _head_kernel(p_ref, cnt_ref, x2_ref, root_ref, bias_ref, batch_ref,
                 gf_ref, gp_w_ref, gp_b_ref, gp_g_ref, gp_be_ref, fc1a_ref,
                 fc1b_ref, fc1_b_ref, fc_g_ref, fc_be_ref, fc2_wt_ref,
                 fc2_b_ref, out_ref):
    inv = 1.0 / jnp.sqrt(1.0 + 1e-5)
    agg = (p_ref[0] + p_ref[1]) / cnt_ref[...]
    x3 = _silu(agg + _mm(x2_ref[...], root_ref[...])
               + bias_ref[0, :][None, :])
    oh = (jax.lax.broadcasted_iota(jnp.int32, (G, N), 0)
          == batch_ref[0, :][None, :]).astype(jnp.float32)
    gcnt = jnp.maximum(jnp.sum(oh, axis=1, keepdims=True), 1.0)
    pooled = _mm(oh, x3) / gcnt
    g = _mm(gf_ref[...], gp_w_ref[...]) + gp_b_ref[...]
    g = _silu(g * inv * gp_g_ref[...] + gp_be_ref[...])
    t = (_mm(pooled, fc1a_ref[...]) + _mm(g, fc1b_ref[...])
         + fc1_b_ref[0, :][None, :])
    t = _silu(t * inv * fc_g_ref[0, :][None, :] + fc_be_ref[0, :][None, :])
    out_ref[...] = (jnp.sum(t * fc2_wt_ref[0, :][None, :], axis=1,
                            keepdims=True) + fc2_b_ref[0, 0])


def kernel(x, edge_index, edge_attr, batch, global_feat, params):
    p = params
    eh = E // 2
    wh = eh // _NW
    srcs = [edge_index[0][h * eh:(h + 1) * eh].reshape(_NW, 1, wh)
            for h in range(2)]
    dsts = [edge_index[1][h * eh:(h + 1) * eh].reshape(_NW, 1, wh)
            for h in range(2)]
    eas = [edge_attr[h * eh:(h + 1) * eh] for h in range(2)]
    dst2d = edge_index[1].reshape(1, E)
    batch2d = batch.reshape(1, N)
    zinit = jnp.zeros((_NC, N, 128), jnp.float32)

    cnt = pl.pallas_call(
        _cnt_kernel,
        grid=(N // NBLK,),
        in_specs=[pl.BlockSpec((1, E), lambda nb: (0, 0))],
        out_specs=pl.BlockSpec((NBLK, 1), lambda nb: (nb, 0)),
        out_shape=jax.ShapeDtypeStruct((N, 1), jnp.float32),
    )(dst2d)
    cnt = jnp.maximum(cnt, 1.0)

    x_pad = jnp.pad(x, ((0, 0), (0, 128 - 9)))
    root1_pad = jnp.pad(p['root1'], ((0, 128 - 9), (0, 0)))

    def layer(x_full, w1, b1, wf, nin):
        msgs = []
        parts = zinit
        for half in range(2):
            xs = _gather(x_full, srcs[half])
            msgs.append(_conv_call(xs, eas[half], w1, b1, wf, nin))
        for half in range(2):
            parts = _scatter_add(msgs[half], dsts[half], parts)
        return parts

    p1 = layer(x_pad, p['ec1_w1'], p['ec1_b1'].reshape(1, 128),
               p['ec1_w2'], 9)
    x1 = _epilogue(p1, cnt, x_pad, root1_pad, p['bias1'].reshape(1, 128))

    p2 = layer(x1, p['ec2_w1'], p['ec2_b1'].reshape(1, 128), p['ec2_w2'], 128)
    x2 = _epilogue(p2, cnt, x1, p['root2'], p['bias2'].reshape(1, 128))

    p3 = layer(x2, p['ec3_w1'], p['ec3_b1'].reshape(1, 128), p['ec3_w2'], 128)

    out = pl.pallas_call(
        _head_kernel,
        in_specs=[
            pl.BlockSpec((_NC, N, 128), lambda: (0, 0, 0)),
            pl.BlockSpec((N, 1), lambda: (0, 0)),
            pl.BlockSpec((N, 128), lambda: (0, 0)),
            pl.BlockSpec((128, 128), lambda: (0, 0)),
            pl.BlockSpec((1, 128), lambda: (0, 0)),
            pl.BlockSpec((1, N), lambda: (0, 0)),
            pl.BlockSpec((1, 10), lambda: (0, 0)),
            pl.BlockSpec((10, 32), lambda: (0, 0)),
            pl.BlockSpec((1, 32), lambda: (0, 0)),
            pl.BlockSpec((1, 32), lambda: (0, 0)),
            pl.BlockSpec((1, 32), lambda: (0, 0)),
            pl.BlockSpec((128, 128), lambda: (0, 0)),
            pl.BlockSpec((32, 128), lambda: (0, 0)),
            pl.BlockSpec((1, 128), lambda: (0, 0)),
            pl.BlockSpec((1, 128), lambda: (0, 0)),
            pl.BlockSpec((1, 128), lambda: (0, 0)),
            pl.BlockSpec((1, 128), lambda: (0, 0)),
            pl.BlockSpec((1, 1), lambda: (0, 0)),
        ],
        out_specs=pl.BlockSpec((G, 1), lambda: (0, 0)),
        out_shape=jax.ShapeDtypeStruct((G, 1), jnp.float32),
    )(p3, cnt, x2, p['root3'], p['bias3'].reshape(1, 128), batch2d,
      global_feat, p['gp_w'], p['gp_b'].reshape(1, 32),
      p['gp_gamma'].reshape(1, 32), p['gp_beta'].reshape(1, 32),
      p['fc1_w'][:128], p['fc1_w'][128:], p['fc1_b'].reshape(1, 128),
      p['fc_gamma'].reshape(1, 128), p['fc_beta'].reshape(1, 128),
      p['fc2_w'].reshape(1, 128), p['fc2_b'].reshape(1, 1))
    return jnp.squeeze(out)


# speedup vs baseline: 1.0248x; 1.0248x over previous
import functools

import jax
import jax.numpy as jnp
from jax.experimental import pallas as pl
from jax.experimental.pallas import tpu as pltpu
from jax.experimental.pallas import tpu_sc as plsc

N, E, G = 4096, 8192, 128
EBLK = 256
NBLK = 256

_NC, _NS = 2, 16
_NW = _NC * _NS
_W = E // _NW
_ROWS = N // _NS

_HI = jax.lax.Precision.HIGHEST


def _silu(v):
    return v * jax.nn.sigmoid(v)


def _mm(a, b, precision=_HI):
    return jnp.dot(a, b, preferred_element_type=jnp.float32,
                   precision=precision)


def _sc_mesh():
    return plsc.VectorSubcoreMesh(core_axis_name="c", subcore_axis_name="s")


def _sc_gather_body(x_hbm, idx_hbm, o_hbm, ibuf, vbuf):
    core = jax.lax.axis_index("c")
    sid = jax.lax.axis_index("s")
    wid = core * _NS + sid
    pltpu.sync_copy(idx_hbm.at[wid], ibuf)
    pltpu.sync_copy(x_hbm.at[ibuf.at[0]], vbuf)
    pltpu.sync_copy(vbuf, o_hbm.at[pl.ds(wid * _W, _W)])


def _gather(x_full, idx3):
    k = pl.kernel(
        _sc_gather_body,
        out_type=jax.ShapeDtypeStruct((E, 128), jnp.float32),
        mesh=_sc_mesh(),
        scratch_types=[pltpu.VMEM((1, _W), jnp.int32),
                       pltpu.VMEM((_W, 128), jnp.float32)],
    )
    return k(x_full, idx3)


def _sc_scatter_body(msg_hbm, idx_hbm, zero_hbm, o_hbm, acc, ibuf, vbuf):
    core = jax.lax.axis_index("c")
    sid = jax.lax.axis_index("s")

    @pl.when(sid == 0)
    def _():
        pltpu.sync_copy(zero_hbm, acc)

    plsc.subcore_barrier()
    wid = core * _NS + sid
    pltpu.sync_copy(idx_hbm.at[wid], ibuf)
    pltpu.sync_copy(msg_hbm.at[pl.ds(wid * _W, _W)], vbuf)
    pltpu.sync_copy(vbuf, acc.at[ibuf.at[0]], add=True)
    plsc.subcore_barrier()
    pltpu.sync_copy(acc.at[pl.ds(sid * _ROWS, _ROWS)],
                    o_hbm.at[core].at[pl.ds(sid * _ROWS, _ROWS)])


def _scatter_add(msg, idx3, zeros_n):
    k = pl.kernel(
        _sc_scatter_body,
        out_type=jax.ShapeDtypeStruct((_NC, N, 128), jnp.float32),
        mesh=_sc_mesh(),
        scratch_types=[pltpu.VMEM_SHARED((N, 128), jnp.float32),
                       pltpu.VMEM((1, _W), jnp.int32),
                       pltpu.VMEM((_W, 128), jnp.float32)],
    )
    return k(msg, idx3, zeros_n)


def _cnt_kernel(dst_ref, out_ref):
    nb = pl.program_id(0)
    row = jax.lax.broadcasted_iota(jnp.int32, (NBLK, E), 0) + nb * NBLK
    eq = (row == dst_ref[0, :][None, :]).astype(jnp.float32)
    out_ref[...] = jnp.sum(eq, axis=1, keepdims=True)


def _conv_kernel(xs_ref, ea_ref, w1_ref, b1_ref, wf_ref, msg_ref, *, nin):
    h = _silu(_mm(ea_ref[...], w1_ref[...]) + b1_ref[0, :][None, :])
    parr = _mm(h, wf_ref[...], jax.lax.Precision.DEFAULT)
    xs = xs_ref[...]
    msg = xs[:, 0:1] * parr[:, 0:128]
    for i in range(1, nin):
        msg += xs[:, i:i + 1] * parr[:, i * 128:(i + 1) * 128]
    msg_ref[...] = msg


def _conv_call(xs, edge_attr, w1, b1, wf, nin):
    n_eb = xs.shape[0] // EBLK
    const = lambda shape: pl.BlockSpec(shape, lambda eb: (0,) * len(shape))
    kfn = functools.partial(_conv_kernel, nin=nin)
    return pl.pallas_call(
        kfn,
        grid=(n_eb,),
        in_specs=[
            pl.BlockSpec((EBLK, 128), lambda eb: (eb, 0)),
            pl.BlockSpec((EBLK, 3), lambda eb: (eb, 0)),
            const(w1.shape), const(b1.shape), const(wf.shape),
        ],
        out_specs=pl.BlockSpec((EBLK, 128), lambda eb: (eb, 0)),
        out_shape=jax.ShapeDtypeStruct((xs.shape[0], 128), jnp.float32),
        compiler_params=pltpu.CompilerParams(
            dimension_semantics=("arbitrary",)),
    )(xs, edge_attr, w1, b1, wf)


def _epi_kernel(p_ref, cnt_ref, x_ref, root_ref, bias_ref, out_ref):
    agg = (p_ref[0] + p_ref[1]) / cnt_ref[...]
    out_ref[...] = _silu(
        agg + _mm(x_ref[...], root_ref[...]) + bias_ref[0, :][None, :])


def _epilogue(partials, cnt, x_full, root, bias):
    return pl.pallas_call(
        _epi_kernel,
        in_specs=[
            pl.BlockSpec((_NC, N, 128), lambda: (0, 0, 0)),
            pl.BlockSpec((N, 1), lambda: (0, 0)),
            pl.BlockSpec(x_full.shape, lambda: (0, 0)),
            pl.BlockSpec((128, 128), lambda: (0, 0)),
            pl.BlockSpec((1, 128), lambda: (0, 0)),
        ],
        out_specs=pl.BlockSpec((N, 128), lambda: (0, 0)),
        out_shape=jax.ShapeDtypeStruct((N, 128), jnp.float32),
    )(partials, cnt, x_full, root, bias)


def _head_kernel(p_ref, cnt_ref, x2_ref, root_ref, bias_ref, batch_ref,
                 gf_ref, gp_w_ref, gp_b_ref, gp_g_ref, gp_be_ref, fc1a_ref,
                 fc1b_ref, fc1_b_ref, fc_g_ref, fc_be_ref, fc2_wt_ref,
                 fc2_b_ref, out_ref):
    inv = 1.0 / jnp.sqrt(1.0 + 1e-5)
    agg = (p_ref[0] + p_ref[1]) / cnt_ref[...]
    x3 = _silu(agg + _mm(x2_ref[...], root_ref[...])
               + bias_ref[0, :][None, :])
    oh = (jax.lax.broadcasted_iota(jnp.int32, (G, N), 0)
          == batch_ref[0, :][None, :]).astype(jnp.float32)
    gcnt = jnp.maximum(jnp.sum(oh, axis=1, keepdims=True), 1.0)
    pooled = _mm(oh, x3) / gcnt
    g = _mm(gf_ref[...], gp_w_ref[...]) + gp_b_ref[...]
    g = _silu(g * inv * gp_g_ref[...] + gp_be_ref[...])
    t = (_mm(pooled, fc1a_ref[...]) + _mm(g, fc1b_ref[...])
         + fc1_b_ref[0, :][None, :])
    t = _silu(t * inv * fc_g_ref[0, :][None, :] + fc_be_ref[0, :][None, :])
    out_ref[...] = (jnp.sum(t * fc2_wt_ref[0, :][None, :], axis=1,
                            keepdims=True) + fc2_b_ref[0, 0])


def kernel(x, edge_index, edge_attr, batch, global_feat, params):
    p = params
    src3 = edge_index[0].reshape(_NW, 1, _W)
    dst3 = edge_index[1].reshape(_NW, 1, _W)
    dst2d = edge_index[1].reshape(1, E)
    batch2d = batch.reshape(1, N)
    zeros_n = jnp.zeros((N, 128), jnp.float32)

    cnt = pl.pallas_call(
        _cnt_kernel,
        grid=(N // NBLK,),
        in_specs=[pl.BlockSpec((1, E), lambda nb: (0, 0))],
        out_specs=pl.BlockSpec((NBLK, 1), lambda nb: (nb, 0)),
        out_shape=jax.ShapeDtypeStruct((N, 1), jnp.float32),
    )(dst2d)
    cnt = jnp.maximum(cnt, 1.0)

    x_pad = jnp.pad(x, ((0, 0), (0, 128 - 9)))
    root1_pad = jnp.pad(p['root1'], ((0, 128 - 9), (0, 0)))

    xs1 = _gather(x_pad, src3)
    msg1 = _conv_call(xs1, edge_attr, p['ec1_w1'],
                      p['ec1_b1'].reshape(1, 128), p['ec1_w2'], 9)
    p1 = _scatter_add(msg1, dst3, zeros_n)
    x1 = _epilogue(p1, cnt, x_pad, root1_pad, p['bias1'].reshape(1, 128))

    xs2 = _gather(x1, src3)
    msg2 = _conv_call(xs2, edge_attr, p['ec2_w1'],
                      p['ec2_b1'].reshape(1, 128), p['ec2_w2'], 128)
    p2 = _scatter_add(msg2, dst3, zeros_n)
    x2 = _epilogue(p2, cnt, x1, p['root2'], p['bias2'].reshape(1, 128))

    xs3 = _gather(x2, src3)
    msg3 = _conv_call(xs3, edge_attr, p['ec3_w1'],
                      p['ec3_b1'].reshape(1, 128), p['ec3_w2'], 128)
    p3 = _scatter_add(msg3, dst3, zeros_n)

    out = pl.pallas_call(
        _head_kernel,
        in_specs=[
            pl.BlockSpec((_NC, N, 128), lambda: (0, 0, 0)),
            pl.BlockSpec((N, 1), lambda: (0, 0)),
            pl.BlockSpec((N, 128), lambda: (0, 0)),
            pl.BlockSpec((128, 128), lambda: (0, 0)),
            pl.BlockSpec((1, 128), lambda: (0, 0)),
            pl.BlockSpec((1, N), lambda: (0, 0)),
            pl.BlockSpec((1, 10), lambda: (0, 0)),
            pl.BlockSpec((10, 32), lambda: (0, 0)),
            pl.BlockSpec((1, 32), lambda: (0, 0)),
            pl.BlockSpec((1, 32), lambda: (0, 0)),
            pl.BlockSpec((1, 32), lambda: (0, 0)),
            pl.BlockSpec((128, 128), lambda: (0, 0)),
            pl.BlockSpec((32, 128), lambda: (0, 0)),
            pl.BlockSpec((1, 128), lambda: (0, 0)),
            pl.BlockSpec((1, 128), lambda: (0, 0)),
            pl.BlockSpec((1, 128), lambda: (0, 0)),
            pl.BlockSpec((1, 128), lambda: (0, 0)),
            pl.BlockSpec((1, 1), lambda: (0, 0)),
        ],
        out_specs=pl.BlockSpec((G, 1), lambda: (0, 0)),
        out_shape=jax.ShapeDtypeStruct((G, 1), jnp.float32),
    )(p3, cnt, x2, p['root3'], p['bias3'].reshape(1, 128), batch2d,
      global_feat, p['gp_w'], p['gp_b'].reshape(1, 32),
      p['gp_gamma'].reshape(1, 32), p['gp_beta'].reshape(1, 32),
      p['fc1_w'][:128], p['fc1_w'][128:], p['fc1_b'].reshape(1, 128),
      p['fc_gamma'].reshape(1, 128), p['fc_beta'].reshape(1, 128),
      p['fc2_w'].reshape(1, 128), p['fc2_b'].reshape(1, 1))
    return jnp.squeeze(out)


# speedup vs baseline: 1.2201x; 1.1905x over previous
import functools

import jax
import jax.numpy as jnp
from jax.experimental import pallas as pl
from jax.experimental.pallas import tpu as pltpu
from jax.experimental.pallas import tpu_sc as plsc

N, E, G = 4096, 8192, 128
EBLK = 256
NBLK = 256

_NC, _NS = 2, 16
_NW = _NC * _NS
_W = E // _NW
_ROWS = N // _NS

_HI = jax.lax.Precision.HIGHEST


def _silu(v):
    return v * jax.nn.sigmoid(v)


def _bf(v):
    return v.astype(jnp.bfloat16).astype(jnp.float32)


def _mm(a, b, precision=jax.lax.Precision.DEFAULT):
    return jnp.dot(a, b, preferred_element_type=jnp.float32,
                   precision=precision)


def _sc_mesh():
    return plsc.VectorSubcoreMesh(core_axis_name="c", subcore_axis_name="s")


def _sc_gather_body(x_hbm, idx_hbm, o_hbm, ibuf, vbuf):
    core = jax.lax.axis_index("c")
    sid = jax.lax.axis_index("s")
    wid = core * _NS + sid
    pltpu.sync_copy(idx_hbm.at[wid], ibuf)
    pltpu.sync_copy(x_hbm.at[ibuf.at[0]], vbuf)
    pltpu.sync_copy(vbuf, o_hbm.at[pl.ds(wid * _W, _W)])


def _gather(x_full, idx3):
    k = pl.kernel(
        _sc_gather_body,
        out_type=jax.ShapeDtypeStruct((E, 128), jnp.float32),
        mesh=_sc_mesh(),
        scratch_types=[pltpu.VMEM((1, _W), jnp.int32),
                       pltpu.VMEM((_W, 128), jnp.float32)],
    )
    return k(x_full, idx3)


def _sc_scatter_body(msg_hbm, idx_hbm, zero_hbm, o_hbm, acc, ibuf, vbuf):
    core = jax.lax.axis_index("c")
    sid = jax.lax.axis_index("s")

    @pl.when(sid == 0)
    def _():
        pltpu.sync_copy(zero_hbm, acc)

    plsc.subcore_barrier()
    wid = core * _NS + sid
    pltpu.sync_copy(idx_hbm.at[wid], ibuf)
    pltpu.sync_copy(msg_hbm.at[pl.ds(wid * _W, _W)], vbuf)
    pltpu.sync_copy(vbuf, acc.at[ibuf.at[0]], add=True)
    plsc.subcore_barrier()
    pltpu.sync_copy(acc.at[pl.ds(sid * _ROWS, _ROWS)],
                    o_hbm.at[core].at[pl.ds(sid * _ROWS, _ROWS)])


def _scatter_add(msg, idx3, zeros_n):
    k = pl.kernel(
        _sc_scatter_body,
        out_type=jax.ShapeDtypeStruct((_NC, N, 128), jnp.float32),
        mesh=_sc_mesh(),
        scratch_types=[pltpu.VMEM_SHARED((N, 128), jnp.float32),
                       pltpu.VMEM((1, _W), jnp.int32),
                       pltpu.VMEM((_W, 128), jnp.float32)],
    )
    return k(msg, idx3, zeros_n)


def _cnt_kernel(dst_ref, out_ref):
    nb = pl.program_id(0)
    row = jax.lax.broadcasted_iota(jnp.int32, (NBLK, E), 0) + nb * NBLK
    eq = (row == dst_ref[0, :][None, :]).astype(jnp.float32)
    out_ref[...] = jnp.sum(eq, axis=1, keepdims=True)


def _conv_kernel(xs_ref, ea_ref, w1_ref, b1_ref, wf_ref, msg_ref, *, nin):
    h = _silu(_mm(ea_ref[...], w1_ref[...]) + b1_ref[0, :][None, :])
    parr = _bf(_mm(h, wf_ref[...]))
    xs = _bf(xs_ref[...])
    msg = xs[:, 0:1] * parr[:, 0:128]
    for i in range(1, nin):
        msg += xs[:, i:i + 1] * parr[:, i * 128:(i + 1) * 128]
    msg_ref[...] = msg


def _conv_call(xs, edge_attr, w1, b1, wf, nin):
    n_eb = xs.shape[0] // EBLK
    const = lambda shape: pl.BlockSpec(shape, lambda eb: (0,) * len(shape))
    kfn = functools.partial(_conv_kernel, nin=nin)
    return pl.pallas_call(
        kfn,
        grid=(n_eb,),
        in_specs=[
            pl.BlockSpec((EBLK, 128), lambda eb: (eb, 0)),
            pl.BlockSpec((EBLK, 3), lambda eb: (eb, 0)),
            const(w1.shape), const(b1.shape), const(wf.shape),
        ],
        out_specs=pl.BlockSpec((EBLK, 128), lambda eb: (eb, 0)),
        out_shape=jax.ShapeDtypeStruct((xs.shape[0], 128), jnp.float32),
        compiler_params=pltpu.CompilerParams(
            dimension_semantics=("arbitrary",)),
    )(xs, edge_attr, w1, b1, wf)


def _epi_kernel(p_ref, cnt_ref, x_ref, root_ref, bias_ref, out_ref):
    agg = (p_ref[0] + p_ref[1]) / cnt_ref[...]
    out_ref[...] = _silu(
        agg + _mm(x_ref[...], root_ref[...]) + bias_ref[0, :][None, :])


def _epilogue(partials, cnt, x_full, root, bias):
    return pl.pallas_call(
        _epi_kernel,
        in_specs=[
            pl.BlockSpec((_NC, N, 128), lambda: (0, 0, 0)),
            pl.BlockSpec((N, 1), lambda: (0, 0)),
            pl.BlockSpec(x_full.shape, lambda: (0, 0)),
            pl.BlockSpec((128, 128), lambda: (0, 0)),
            pl.BlockSpec((1, 128), lambda: (0, 0)),
        ],
        out_specs=pl.BlockSpec((N, 128), lambda: (0, 0)),
        out_shape=jax.ShapeDtypeStruct((N, 128), jnp.float32),
    )(partials, cnt, x_full, root, bias)


def _head_kernel(p_ref, cnt_ref, x2_ref, root_ref, bias_ref, batch_ref,
                 gf_ref, gp_w_ref, gp_b_ref, gp_g_ref, gp_be_ref, fc1a_ref,
                 fc1b_ref, fc1_b_ref, fc_g_ref, fc_be_ref, fc2_wt_ref,
                 fc2_b_ref, out_ref):
    inv = 1.0 / jnp.sqrt(1.0 + 1e-5)
    agg = (p_ref[0] + p_ref[1]) / cnt_ref[...]
    x3 = _silu(agg + _mm(x2_ref[...], root_ref[...])
               + bias_ref[0, :][None, :])
    oh = (jax.lax.broadcasted_iota(jnp.int32, (G, N), 0)
          == batch_ref[0, :][None, :]).astype(jnp.float32)
    gcnt = jnp.maximum(jnp.sum(oh, axis=1, keepdims=True), 1.0)
    pooled = _mm(oh, x3, _HI) / gcnt
    g = _mm(gf_ref[...], gp_w_ref[...]) + gp_b_ref[...]
    g = _silu(g * inv * gp_g_ref[...] + gp_be_ref[...])
    t = (_mm(pooled, fc1a_ref[...]) + _mm(g, fc1b_ref[...])
         + fc1_b_ref[0, :][None, :])
    t = _silu(t * inv * fc_g_ref[0, :][None, :] + fc_be_ref[0, :][None, :])
    out_ref[...] = (jnp.sum(_bf(t) * _bf(fc2_wt_ref[0, :][None, :]), axis=1,
                            keepdims=True) + fc2_b_ref[0, 0])


def kernel(x, edge_index, edge_attr, batch, global_feat, params):
    p = params
    src3 = edge_index[0].reshape(_NW, 1, _W)
    dst3 = edge_index[1].reshape(_NW, 1, _W)
    dst2d = edge_index[1].reshape(1, E)
    batch2d = batch.reshape(1, N)
    zeros_n = jnp.zeros((N, 128), jnp.float32)

    cnt = pl.pallas_call(
        _cnt_kernel,
        grid=(N // NBLK,),
        in_specs=[pl.BlockSpec((1, E), lambda nb: (0, 0))],
        out_specs=pl.BlockSpec((NBLK, 1), lambda nb: (nb, 0)),
        out_shape=jax.ShapeDtypeStruct((N, 1), jnp.float32),
    )(dst2d)
    cnt = jnp.maximum(cnt, 1.0)

    x_pad = jnp.pad(x, ((0, 0), (0, 128 - 9)))
    root1_pad = jnp.pad(p['root1'], ((0, 128 - 9), (0, 0)))

    xs1 = _gather(x_pad, src3)
    msg1 = _conv_call(xs1, edge_attr, p['ec1_w1'],
                      p['ec1_b1'].reshape(1, 128), p['ec1_w2'], 9)
    p1 = _scatter_add(msg1, dst3, zeros_n)
    x1 = _epilogue(p1, cnt, x_pad, root1_pad, p['bias1'].reshape(1, 128))

    xs2 = _gather(x1, src3)
    msg2 = _conv_call(xs2, edge_attr, p['ec2_w1'],
                      p['ec2_b1'].reshape(1, 128), p['ec2_w2'], 128)
    p2 = _scatter_add(msg2, dst3, zeros_n)
    x2 = _epilogue(p2, cnt, x1, p['root2'], p['bias2'].reshape(1, 128))

    xs3 = _gather(x2, src3)
    msg3 = _conv_call(xs3, edge_attr, p['ec3_w1'],
                      p['ec3_b1'].reshape(1, 128), p['ec3_w2'], 128)
    p3 = _scatter_add(msg3, dst3, zeros_n)

    out = pl.pallas_call(
        _head_kernel,
        in_specs=[
            pl.BlockSpec((_NC, N, 128), lambda: (0, 0, 0)),
            pl.BlockSpec((N, 1), lambda: (0, 0)),
            pl.BlockSpec((N, 128), lambda: (0, 0)),
            pl.BlockSpec((128, 128), lambda: (0, 0)),
            pl.BlockSpec((1, 128), lambda: (0, 0)),
            pl.BlockSpec((1, N), lambda: (0, 0)),
            pl.BlockSpec((1, 10), lambda: (0, 0)),
            pl.BlockSpec((10, 32), lambda: (0, 0)),
            pl.BlockSpec((1, 32), lambda: (0, 0)),
            pl.BlockSpec((1, 32), lambda: (0, 0)),
            pl.BlockSpec((1, 32), lambda: (0, 0)),
            pl.BlockSpec((128, 128), lambda: (0, 0)),
            pl.BlockSpec((32, 128), lambda: (0, 0)),
            pl.BlockSpec((1, 128), lambda: (0, 0)),
            pl.BlockSpec((1, 128), lambda: (0, 0)),
            pl.BlockSpec((1, 128), lambda: (0, 0)),
            pl.BlockSpec((1, 128), lambda: (0, 0)),
            pl.BlockSpec((1, 1), lambda: (0, 0)),
        ],
        out_specs=pl.BlockSpec((G, 1), lambda: (0, 0)),
        out_shape=jax.ShapeDtypeStruct((G, 1), jnp.float32),
    )(p3, cnt, x2, p['root3'], p['bias3'].reshape(1, 128), batch2d,
      global_feat, p['gp_w'], p['gp_b'].reshape(1, 32),
      p['gp_gamma'].reshape(1, 32), p['gp_beta'].reshape(1, 32),
      p['fc1_w'][:128], p['fc1_w'][128:], p['fc1_b'].reshape(1, 128),
      p['fc_gamma'].reshape(1, 128), p['fc_beta'].reshape(1, 128),
      p['fc2_w'].reshape(1, 128), p['fc2_b'].reshape(1, 1))
    return jnp.squeeze(out)
